# baseline (speedup 1.0000x reference)
import functools
import math

import jax
import jax.numpy as jnp
from jax import lax
from jax.experimental import pallas as pl
from jax.experimental.pallas import tpu as pltpu
from jax.experimental.pallas import tpu_sc as plsc

_NUM_BUCKETS = 64
_MAX_DISTANCE = 256
_OUT_DIM = 16
_SEQ = 2048

_NC = 2
_NS = 16
_NW = _NC * _NS
_ROWS_W = _SEQ // _NW
_NT = _SEQ // 128


def _table_body(w_ref, t_ref):
    half = _NUM_BUCKETS // 2
    max_exact = half // 2
    v = lax.broadcasted_iota(jnp.int32, (_SEQ, _NUM_BUCKETS), 0)
    col = lax.broadcasted_iota(jnp.int32, (_SEQ, _NUM_BUCKETS), 1)
    val_large = max_exact + (
        jnp.log(v / max_exact)
        / math.log(_MAX_DISTANCE / max_exact)
        * (half - max_exact)
    ).astype(jnp.int32)
    val_large = jnp.minimum(val_large, jnp.full_like(val_large, half - 1))
    bucket = jnp.where(v < max_exact, v, val_large)
    onehot = (bucket == col).astype(jnp.float32)
    t_ref[...] = jnp.dot(onehot, w_ref[...], preferred_element_type=jnp.float32)


def _build_table(w):
    return pl.pallas_call(
        _table_body,
        out_shape=jax.ShapeDtypeStruct((_SEQ, _OUT_DIM), jnp.float32),
    )(w)


@functools.partial(
    pl.kernel,
    out_type=jax.ShapeDtypeStruct((_SEQ, 2 * _NT, 8, 128), jnp.float32),
    mesh=plsc.VectorSubcoreMesh(core_axis_name="c", subcore_axis_name="s"),
    compiler_params=pltpu.CompilerParams(
        use_tc_tiling_on_sc=True, needs_layout_passes=False
    ),
    scratch_types=[
        pltpu.VMEM((32, 8, 128), jnp.float32),
        pltpu.VMEM((1, _NT, 8, 128), jnp.int32),
        pltpu.VMEM((2, 1, 2 * _NT, 8, 128), jnp.float32),
        pltpu.SemaphoreType.DMA,
        pltpu.SemaphoreType.DMA,
        pltpu.SemaphoreType.DMA,
    ],
)
def _sc_gather(t_hbm, rp_hbm, out_hbm, t_v, idx_v, obuf, sem_i, sem_o0, sem_o1):
    wid = lax.axis_index("s") * _NC + lax.axis_index("c")
    tile_row0 = wid * (_ROWS_W // 8)
    sem_o = (sem_o0, sem_o1)

    def wait_store(b):
        pltpu.make_async_copy(
            obuf.at[b], out_hbm.at[pl.ds(0, 1)], sem_o[b]
        ).wait()

    pltpu.sync_copy(t_hbm, t_v)

    def chunk(a, c):
        it = tile_row0 + a
        pltpu.async_copy(rp_hbm.at[pl.ds(it, 1)], idx_v, sem_i)
        pltpu.make_async_copy(rp_hbm.at[pl.ds(0, 1)], idx_v, sem_i).wait()

        def pair(p, cc):
            for b in (0, 1):
                r = p * 2 + b

                @pl.when(a * 8 + r >= 2)
                def _():
                    wait_store(b)

                @plsc.parallel_loop(0, _SEQ // 16, 1, unroll=4)
                def _(jv):
                    jt = jv >> 3
                    c8 = (jv & 7) << 4
                    jvec = idx_v[0, jt, r, pl.ds(c8, 16)]
                    rt = jvec >> 6
                    rr = (jvec >> 3) & 7
                    colb = (jvec & 7) << 4
                    for d in range(_OUT_DIM):
                        g = plsc.load_gather(t_v, [rt, rr, colb + d])
                        obuf[b, 0, (d // 8) * _NT + jt, d % 8, pl.ds(c8, 16)] = g
                pltpu.async_copy(
                    obuf.at[b], out_hbm.at[pl.ds(it * 8 + r, 1)], sem_o[b]
                )
            return cc

        lax.fori_loop(0, 4, pair, 0)
        return c

    lax.fori_loop(0, _ROWS_W // 8, chunk, 0)
    wait_store(0)
    wait_store(1)


def kernel(relative_position, W):
    t = _build_table(W)
    t3 = t.reshape(32, 8, 128)
    rp4 = relative_position.reshape(_SEQ // 8, 8, _NT, 128).transpose(0, 2, 1, 3)
    out4 = _sc_gather(t3, rp4)
    out = (
        out4.reshape(_SEQ, 2, _NT, 8, 128)
        .transpose(0, 2, 4, 1, 3)
        .reshape(_SEQ, _SEQ, _OUT_DIM)
    )
    return out


# speedup vs baseline: 2.4386x; 2.4386x over previous
import functools
import math

import jax
import jax.numpy as jnp
from jax import lax
from jax.experimental import pallas as pl
from jax.experimental.pallas import tpu as pltpu
from jax.experimental.pallas import tpu_sc as plsc

_NUM_BUCKETS = 64
_MAX_DISTANCE = 256
_OUT_DIM = 16
_SEQ = 2048

_NC = 2
_NS = 16
_NW = _NC * _NS
_ROWS_W = _SEQ // _NW
_NT = _SEQ // 128


def _bucket_body(o_ref):
    half = _NUM_BUCKETS // 2
    max_exact = half // 2
    r = lax.broadcasted_iota(jnp.int32, (16, 128), 0)
    c = lax.broadcasted_iota(jnp.int32, (16, 128), 1)
    v = r * 128 + c
    val_large = max_exact + (
        jnp.log(v / max_exact)
        / math.log(_MAX_DISTANCE / max_exact)
        * (half - max_exact)
    ).astype(jnp.int32)
    val_large = jnp.minimum(val_large, jnp.full_like(val_large, half - 1))
    bucket = jnp.where(v < max_exact, v, val_large)
    o_ref[...] = bucket << 8


def _build_b256():
    return pl.pallas_call(
        _bucket_body,
        out_shape=jax.ShapeDtypeStruct((16, 128), jnp.int32),
    )()


@functools.partial(
    pl.kernel,
    out_type=jax.ShapeDtypeStruct((_SEQ, 2 * _NT, 8, 128), jnp.float32),
    mesh=plsc.VectorSubcoreMesh(core_axis_name="c", subcore_axis_name="s"),
    compiler_params=pltpu.CompilerParams(
        use_tc_tiling_on_sc=True, needs_layout_passes=False
    ),
    scratch_types=[
        pltpu.VMEM((64, 128), jnp.float32),
        pltpu.VMEM((8192,), jnp.float32),
        pltpu.VMEM((2, 8, 128), jnp.int32),
        pltpu.VMEM((2048,), jnp.int32),
        pltpu.VMEM((1, _NT, 8, 128), jnp.int32),
        pltpu.VMEM((2, 1, 2 * _NT, 8, 128), jnp.float32),
        pltpu.SemaphoreType.DMA,
        pltpu.SemaphoreType.DMA,
        pltpu.SemaphoreType.DMA,
    ],
)
def _sc_gather(
    t_hbm, b_hbm, rp_hbm, out_hbm,
    t2d, t1, b_s, b1, idx_v, obuf,
    sem_i, sem_o0, sem_o1,
):
    wid = lax.axis_index("s") * _NC + lax.axis_index("c")
    tile_row0 = wid * (_ROWS_W // 8)
    sem_o = (sem_o0, sem_o1)

    def wait_store(b):
        pltpu.make_async_copy(
            obuf.at[b], out_hbm.at[pl.ds(0, 1)], sem_o[b]
        ).wait()

    pltpu.sync_copy(t_hbm, t2d)
    pltpu.sync_copy(b_hbm, b_s)

    def flat_t(k, c):
        t1[pl.ds(k * 16, 16)] = t2d[k >> 3, pl.ds((k & 7) * 16, 16)]
        return c

    lax.fori_loop(0, 512, flat_t, 0)

    def flat_b(k, c):
        b1[pl.ds(k * 16, 16)] = b_s[k >> 6, (k >> 3) & 7, pl.ds((k & 7) * 16, 16)]
        return c

    lax.fori_loop(0, 128, flat_b, 0)

    lane = lax.iota(jnp.int32, 16)

    def chunk(a, c):
        it = tile_row0 + a
        pltpu.async_copy(rp_hbm.at[pl.ds(it, 1)], idx_v, sem_i)
        pltpu.make_async_copy(rp_hbm.at[pl.ds(0, 1)], idx_v, sem_i).wait()

        def pair(p, cc):
            for b in (0, 1):
                r = p * 2 + b

                @pl.when(a * 8 + r >= 2)
                def _():
                    wait_store(b)

                @plsc.parallel_loop(0, _SEQ // 16, 1, unroll=2)
                def _(jv):
                    jt = jv >> 3
                    c8 = (jv & 7) << 4
                    jvec = idx_v[0, jt, r, pl.ds(c8, 16)]
                    w = plsc.load_gather(b1, [jvec])
                    wl = w + lane
                    for d in range(_OUT_DIM):
                        g = plsc.load_gather(t1, [wl + d * 16])
                        obuf[b, 0, (d // 8) * _NT + jt, d % 8, pl.ds(c8, 16)] = g

                pltpu.async_copy(
                    obuf.at[b], out_hbm.at[pl.ds(it * 8 + r, 1)], sem_o[b]
                )
            return cc

        lax.fori_loop(0, 4, pair, 0)
        return c

    lax.fori_loop(0, _ROWS_W // 8, chunk, 0)
    wait_store(0)
    wait_store(1)


def kernel(relative_position, W):
    b256 = _build_b256().reshape(2, 8, 128)
    t_rep = jnp.broadcast_to(W[:32, :, None], (32, 16, 16)).reshape(64, 128)
    rp4 = relative_position.reshape(_SEQ // 8, 8, _NT, 128).transpose(0, 2, 1, 3)
    out4 = _sc_gather(t_rep, b256, rp4)
    out = (
        out4.reshape(_SEQ, 2, _NT, 8, 128)
        .transpose(0, 2, 4, 1, 3)
        .reshape(_SEQ, _SEQ, _OUT_DIM)
    )
    return out


# speedup vs baseline: 3.1349x; 1.2855x over previous
import functools
import math

import jax
import jax.numpy as jnp
from jax import lax
from jax.experimental import pallas as pl
from jax.experimental.pallas import tpu as pltpu
from jax.experimental.pallas import tpu_sc as plsc

_NUM_BUCKETS = 64
_MAX_DISTANCE = 256
_OUT_DIM = 16
_SEQ = 2048

_NC = 2
_NS = 16
_NW = _NC * _NS
_ROWS_W = _SEQ // _NW
_NT = _SEQ // 128


def _bucket_body(o_ref):
    half = _NUM_BUCKETS // 2
    max_exact = half // 2
    r = lax.broadcasted_iota(jnp.int32, (16, 128), 0)
    c = lax.broadcasted_iota(jnp.int32, (16, 128), 1)
    v = r * 128 + c
    val_large = max_exact + (
        jnp.log(v / max_exact)
        / math.log(_MAX_DISTANCE / max_exact)
        * (half - max_exact)
    ).astype(jnp.int32)
    val_large = jnp.minimum(val_large, jnp.full_like(val_large, half - 1))
    bucket = jnp.where(v < max_exact, v, val_large)
    o_ref[...] = bucket << 8


def _build_b256():
    return pl.pallas_call(
        _bucket_body,
        out_shape=jax.ShapeDtypeStruct((16, 128), jnp.int32),
    )()


@functools.partial(
    pl.kernel,
    out_type=jax.ShapeDtypeStruct((_SEQ, 2 * _NT, 8, 128), jnp.float32),
    mesh=plsc.VectorSubcoreMesh(core_axis_name="c", subcore_axis_name="s"),
    compiler_params=pltpu.CompilerParams(
        use_tc_tiling_on_sc=True, needs_layout_passes=False
    ),
    scratch_types=[
        pltpu.VMEM((64, 128), jnp.float32),
        pltpu.VMEM((8192,), jnp.float32),
        pltpu.VMEM((2, 8, 128), jnp.int32),
        pltpu.VMEM((2048,), jnp.int32),
        pltpu.VMEM((2, 1, _NT, 8, 128), jnp.int32),
        pltpu.VMEM((2, 1, 2 * _NT, 8, 128), jnp.float32),
        pltpu.SemaphoreType.DMA,
        pltpu.SemaphoreType.DMA,
        pltpu.SemaphoreType.DMA,
    ],
)
def _sc_gather(
    t_hbm, b_hbm, rp_hbm, out_hbm,
    t2d, t1, b_s, b1, idx_v, obuf,
    sem_i, sem_o0, sem_o1,
):
    wid = lax.axis_index("s") * _NC + lax.axis_index("c")
    tile_row0 = wid * (_ROWS_W // 8)
    sem_o = (sem_o0, sem_o1)

    def wait_store(b):
        pltpu.make_async_copy(
            obuf.at[b], out_hbm.at[pl.ds(0, 1)], sem_o[b]
        ).wait()

    pltpu.sync_copy(t_hbm, t2d)
    pltpu.sync_copy(b_hbm, b_s)

    def flat_t(k, c):
        t1[pl.ds(k * 16, 16)] = t2d[k >> 3, pl.ds((k & 7) * 16, 16)]
        return c

    lax.fori_loop(0, 512, flat_t, 0)

    def flat_b(k, c):
        b1[pl.ds(k * 16, 16)] = b_s[k >> 6, (k >> 3) & 7, pl.ds((k & 7) * 16, 16)]
        return c

    lax.fori_loop(0, 128, flat_b, 0)

    lane = lax.iota(jnp.int32, 16)

    def issue_idx(a, ib):
        pltpu.async_copy(
            rp_hbm.at[pl.ds(tile_row0 + a, 1)], idx_v.at[ib], sem_i
        )

    def wait_idx(ib):
        pltpu.make_async_copy(
            rp_hbm.at[pl.ds(0, 1)], idx_v.at[ib], sem_i
        ).wait()

    issue_idx(0, 0)

    def superchunk(a2, c):
        for ib in (0, 1):
            a = a2 * 2 + ib
            it = tile_row0 + a
            wait_idx(ib)

            @pl.when(a + 1 < _ROWS_W // 8)
            def _():
                issue_idx(a + 1, 1 - ib)

            def pair(p, cc):
                for b in (0, 1):
                    r = p * 2 + b

                    @pl.when(a * 8 + r >= 2)
                    def _():
                        wait_store(b)

                    @plsc.parallel_loop(0, _SEQ // 16, 1, unroll=2)
                    def _(jv):
                        jt = jv >> 3
                        c8 = (jv & 7) << 4
                        jvec = idx_v[ib, 0, jt, r, pl.ds(c8, 16)]
                        w = plsc.load_gather(b1, [jvec])
                        wl = w + lane
                        for d in range(_OUT_DIM):
                            g = plsc.load_gather(t1, [wl + d * 16])
                            obuf[b, 0, (d // 8) * _NT + jt, d % 8, pl.ds(c8, 16)] = g

                    pltpu.async_copy(
                        obuf.at[b], out_hbm.at[pl.ds(it * 8 + r, 1)], sem_o[b]
                    )
                return cc

            lax.fori_loop(0, 4, pair, 0)
        return c

    lax.fori_loop(0, _ROWS_W // 16, superchunk, 0)
    wait_store(0)
    wait_store(1)


def kernel(relative_position, W):
    b256 = _build_b256().reshape(2, 8, 128)
    t_rep = jnp.broadcast_to(W[:32, :, None], (32, 16, 16)).reshape(64, 128)
    rp4 = relative_position.reshape(_SEQ // 8, 8, _NT, 128).transpose(0, 2, 1, 3)
    out4 = _sc_gather(t_rep, b256, rp4)
    out = (
        out4.reshape(_SEQ, 2, _NT, 8, 128)
        .transpose(0, 2, 4, 1, 3)
        .reshape(_SEQ, _SEQ, _OUT_DIM)
    )
    return out
